# baseline (speedup 1.0000x reference)
import jax
import jax.numpy as jnp
from jax.experimental import pallas as pl
from jax.experimental.pallas import tpu as pltpu

import numpy as np

_K = 64


def _enc_kernel(h_ref, w_ref, b_ref, out_ref):
    acc = jax.lax.dot_general(h_ref[...], w_ref[...], (((1,), (1,)), ((), ())),
                              preferred_element_type=jnp.float32)
    out_ref[...] = acc + b_ref[...]


def _topk_kernel(zp_ref, z_ref, zbf_ref, key_ref):
    x = zp_ref[...]
    ik = jax.lax.bitcast_convert_type(x, jnp.int32)
    key = jnp.where(ik >= 0, ik,
                    jnp.bitwise_xor(jnp.bitwise_not(ik), np.int32(-2147483648)))
    key_ref[...] = key
    kmin = jnp.min(key, axis=1, keepdims=True)
    kmax = jnp.max(key, axis=1, keepdims=True)
    probe = jnp.where(kmax >= np.int32(-2147483648 + 8388608),
                      kmax - np.int32(8388608), kmin)
    cnt_p = jnp.sum((key >= probe).astype(jnp.int32), axis=1, keepdims=True)
    use_p = cnt_p >= _K
    lo0 = jnp.where(use_p, probe, kmin)
    hi0 = kmax + 1
    clo0 = jnp.where(use_p, cnt_p, np.int32(0x7fffffff))
    chi0 = jnp.zeros_like(lo0)
    t0 = lo0
    found0 = jnp.zeros_like(lo0)
    log_k = float(np.log(float(_K)))

    def cond(c):
        it, lo, hi, t, found, clo, chi = c
        return jnp.logical_and(it < 40, jnp.min(found) == 0)

    def body(c):
        it, lo, hi, t, found, clo, chi = c
        mid_b = (lo >> 1) + (hi >> 1) + (lo & hi & 1)
        cl = jnp.log(jnp.minimum(clo, 16777216).astype(jnp.float32))
        ch = jnp.log(jnp.maximum(chi.astype(jnp.float32), 0.5))
        r = jnp.clip((cl - log_k) / (cl - ch), 0.05, 0.95)
        span = hi.astype(jnp.float32) - lo.astype(jnp.float32)
        mid_s = lo + (span * r).astype(jnp.int32)
        mid = jnp.where(it % 2 == 0, mid_s, mid_b)
        mid = jnp.clip(mid, lo + 1, hi - 1)
        cnt = jnp.sum((key_ref[...] >= mid).astype(jnp.int32), axis=1,
                      keepdims=True)
        ge = cnt >= _K
        active = found == 0
        upd_lo = jnp.logical_and(active, ge)
        upd_hi = jnp.logical_and(active, jnp.logical_not(ge))
        lo = jnp.where(upd_lo, mid, lo)
        clo = jnp.where(upd_lo, cnt, clo)
        hi = jnp.where(upd_hi, mid, hi)
        chi = jnp.where(upd_hi, cnt, chi)
        hit = jnp.logical_or(cnt == _K,
                             jnp.logical_or(hi == lo + 1, hi == lo))
        newly = jnp.logical_and(active, hit)
        t = jnp.where(newly, lo, t)
        found = jnp.where(newly, 1, found)
        return it + 1, lo, hi, t, found, clo, chi

    _, lo_f, _, t, found_f, _, _ = jax.lax.while_loop(
        cond, body, (jnp.int32(0), lo0, hi0, t0, found0, clo0, chi0))
    t = jnp.where(found_f == 0, lo_f, t)
    z = jnp.where(key_ref[...] >= t, jnp.maximum(x, 0.0), 0.0)
    z_ref[...] = z
    zbf_ref[...] = z.astype(jnp.bfloat16)


def _dec_kernel(z_ref, w_ref, b_ref, out_ref):
    acc = jax.lax.dot_general(z_ref[...], w_ref[...], (((1,), (1,)), ((), ())),
                              preferred_element_type=jnp.float32)
    out_ref[...] = acc + b_ref[...]


def kernel(h, W_enc, b_enc, W_dec, b_dec):
    N, D = h.shape
    L = W_enc.shape[0]
    h_bf = h.astype(jnp.bfloat16)
    We_bf = W_enc.astype(jnp.bfloat16)
    Wd_bf = W_dec.astype(jnp.bfloat16)
    be2 = b_enc.reshape(1, L)
    bd2 = b_dec.reshape(1, D)

    BM, BN = min(512, N), min(2048, L)
    z_pre = pl.pallas_call(
        _enc_kernel,
        grid=(L // BN, N // BM),
        in_specs=[
            pl.BlockSpec((BM, D), lambda j, i: (i, 0)),
            pl.BlockSpec((BN, D), lambda j, i: (j, 0)),
            pl.BlockSpec((1, BN), lambda j, i: (0, j)),
        ],
        out_specs=pl.BlockSpec((BM, BN), lambda j, i: (i, j)),
        out_shape=jax.ShapeDtypeStruct((N, L), jnp.float32),
        compiler_params=pltpu.CompilerParams(
            dimension_semantics=("parallel", "parallel")),
    )(h_bf, We_bf, be2)

    BR = min(128, N)
    z, z_bf = pl.pallas_call(
        _topk_kernel,
        grid=(N // BR,),
        in_specs=[pl.BlockSpec((BR, L), lambda i: (i, 0))],
        out_specs=[pl.BlockSpec((BR, L), lambda i: (i, 0)),
                   pl.BlockSpec((BR, L), lambda i: (i, 0))],
        out_shape=[jax.ShapeDtypeStruct((N, L), jnp.float32),
                   jax.ShapeDtypeStruct((N, L), jnp.bfloat16)],
        scratch_shapes=[pltpu.VMEM((BR, L), jnp.int32)],
        compiler_params=pltpu.CompilerParams(
            dimension_semantics=("parallel",)),
    )(z_pre)

    BMD, BC = min(256, N), min(512, D)
    h_hat = pl.pallas_call(
        _dec_kernel,
        grid=(D // BC, N // BMD),
        in_specs=[
            pl.BlockSpec((BMD, L), lambda c, i: (i, 0)),
            pl.BlockSpec((BC, L), lambda c, i: (c, 0)),
            pl.BlockSpec((1, BC), lambda c, i: (0, c)),
        ],
        out_specs=pl.BlockSpec((BMD, BC), lambda c, i: (i, c)),
        out_shape=jax.ShapeDtypeStruct((N, D), jnp.float32),
        compiler_params=pltpu.CompilerParams(
            dimension_semantics=("parallel", "parallel")),
    )(z_bf, Wd_bf, bd2)
    return (h_hat, z)


# speedup vs baseline: 1.0278x; 1.0278x over previous
import jax
import jax.numpy as jnp
from jax.experimental import pallas as pl
from jax.experimental.pallas import tpu as pltpu

import numpy as np

_K = 64


def _enc_kernel(h_ref, w_ref, b_ref, out_ref):
    acc = jax.lax.dot_general(h_ref[...], w_ref[...], (((1,), (1,)), ((), ())),
                              preferred_element_type=jnp.float32)
    out_ref[...] = acc + b_ref[...]


def _topk_kernel(zp_ref, z_ref, zbf_ref, key_ref):
    x = zp_ref[...]
    ik = jax.lax.bitcast_convert_type(x, jnp.int32)
    key = jnp.where(ik >= 0, ik,
                    jnp.bitwise_xor(jnp.bitwise_not(ik), np.int32(-2147483648)))
    key_ref[...] = key
    kmin = jnp.min(key, axis=1, keepdims=True)
    kmax = jnp.max(key, axis=1, keepdims=True)
    probe = jnp.where(kmax >= np.int32(-2147483648 + 8388608),
                      kmax - np.int32(8388608), kmin)
    cnt_p = jnp.sum((key >= probe).astype(jnp.int32), axis=1, keepdims=True)
    use_p = cnt_p >= _K
    lo0 = jnp.where(use_p, probe, kmin)
    hi0 = kmax + 1
    t0 = lo0
    found0 = jnp.zeros_like(lo0)

    def cond(c):
        it, lo, hi, t, found = c
        return jnp.logical_and(it < 40, jnp.min(found) == 0)

    def body(c):
        it, lo, hi, t, found = c
        mid = (lo >> 1) + (hi >> 1) + (lo & hi & 1)
        cnt = jnp.sum((key_ref[...] >= mid).astype(jnp.int32), axis=1,
                      keepdims=True)
        ge = cnt >= _K
        active = found == 0
        lo = jnp.where(jnp.logical_and(active, ge), mid, lo)
        hi = jnp.where(jnp.logical_and(active, jnp.logical_not(ge)), mid, hi)
        hit = jnp.logical_or(cnt == _K,
                             jnp.logical_or(hi == lo + 1, hi == lo))
        newly = jnp.logical_and(active, hit)
        t = jnp.where(newly, lo, t)
        found = jnp.where(newly, 1, found)
        return it + 1, lo, hi, t, found

    _, lo_f, _, t, found_f = jax.lax.while_loop(
        cond, body, (jnp.int32(0), lo0, hi0, t0, found0))
    t = jnp.where(found_f == 0, lo_f, t)
    z = jnp.where(key_ref[...] >= t, jnp.maximum(x, 0.0), 0.0)
    z_ref[...] = z
    zbf_ref[...] = z.astype(jnp.bfloat16)


def _dec_kernel(z_ref, w_ref, b_ref, out_ref):
    acc = jax.lax.dot_general(z_ref[...], w_ref[...], (((1,), (1,)), ((), ())),
                              preferred_element_type=jnp.float32)
    out_ref[...] = acc + b_ref[...]


def kernel(h, W_enc, b_enc, W_dec, b_dec):
    N, D = h.shape
    L = W_enc.shape[0]
    h_bf = h.astype(jnp.bfloat16)
    We_bf = W_enc.astype(jnp.bfloat16)
    Wd_bf = W_dec.astype(jnp.bfloat16)
    be2 = b_enc.reshape(1, L)
    bd2 = b_dec.reshape(1, D)

    BM, BN = min(512, N), min(2048, L)
    z_pre = pl.pallas_call(
        _enc_kernel,
        grid=(L // BN, N // BM),
        in_specs=[
            pl.BlockSpec((BM, D), lambda j, i: (i, 0)),
            pl.BlockSpec((BN, D), lambda j, i: (j, 0)),
            pl.BlockSpec((1, BN), lambda j, i: (0, j)),
        ],
        out_specs=pl.BlockSpec((BM, BN), lambda j, i: (i, j)),
        out_shape=jax.ShapeDtypeStruct((N, L), jnp.float32),
        compiler_params=pltpu.CompilerParams(
            dimension_semantics=("parallel", "parallel")),
    )(h_bf, We_bf, be2)

    BR = min(128, N)
    z, z_bf = pl.pallas_call(
        _topk_kernel,
        grid=(N // BR,),
        in_specs=[pl.BlockSpec((BR, L), lambda i: (i, 0))],
        out_specs=[pl.BlockSpec((BR, L), lambda i: (i, 0)),
                   pl.BlockSpec((BR, L), lambda i: (i, 0))],
        out_shape=[jax.ShapeDtypeStruct((N, L), jnp.float32),
                   jax.ShapeDtypeStruct((N, L), jnp.bfloat16)],
        scratch_shapes=[pltpu.VMEM((BR, L), jnp.int32)],
        compiler_params=pltpu.CompilerParams(
            dimension_semantics=("parallel",)),
    )(z_pre)

    BMD, BC = min(256, N), min(512, D)
    h_hat = pl.pallas_call(
        _dec_kernel,
        grid=(D // BC, N // BMD),
        in_specs=[
            pl.BlockSpec((BMD, L), lambda c, i: (i, 0)),
            pl.BlockSpec((BC, L), lambda c, i: (c, 0)),
            pl.BlockSpec((1, BC), lambda c, i: (0, c)),
        ],
        out_specs=pl.BlockSpec((BMD, BC), lambda c, i: (i, c)),
        out_shape=jax.ShapeDtypeStruct((N, D), jnp.float32),
        compiler_params=pltpu.CompilerParams(
            dimension_semantics=("parallel", "parallel")),
    )(z_bf, Wd_bf, bd2)
    return (h_hat, z)
